# baseline (speedup 1.0000x reference)
import functools

import jax
import jax.numpy as jnp
from jax import lax
from jax.experimental import pallas as pl
from jax.experimental.pallas import tpu as pltpu
from jax.experimental.pallas import tpu_sc as plsc

EMBED = 768
SEQ = 2048
BATCH = 4
NW = 32
POSW = SEQ // NW
WIN = 16
NWIN = POSW // WIN
DEPTH = 2
LANES = 16
EMB_VECS = EMBED // LANES

_mesh = plsc.VectorSubcoreMesh(core_axis_name="c", subcore_axis_name="s")


@functools.partial(
    pl.kernel,
    out_type=jax.ShapeDtypeStruct((BATCH, SEQ, EMBED), jnp.float32),
    mesh=_mesh,
    scratch_types=[
        pltpu.VMEM((NWIN, BATCH * WIN), jnp.int32),
        pltpu.VMEM((DEPTH, BATCH * WIN, EMBED), jnp.float32),
        pltpu.VMEM((DEPTH, WIN, EMBED), jnp.float32),
        pltpu.SemaphoreType.DMA,
        (pltpu.SemaphoreType.DMA,) * DEPTH,
        (pltpu.SemaphoreType.DMA,) * DEPTH,
    ],
)
def _embed_add(ids_hbm, wte_hbm, wpe_hbm, out_hbm,
               idx_v, tok_v, pos_v, sem_idx, sem_in, sem_out):
    wid = lax.axis_index("s") * 2 + lax.axis_index("c")
    p0 = wid * POSW

    idx_copies = [
        pltpu.async_copy(ids_hbm.at[b, pl.ds(p0 + w * WIN, WIN)],
                         idx_v.at[w, pl.ds(b * WIN, WIN)], sem_idx)
        for w in range(NWIN) for b in range(BATCH)
    ]
    for cp in idx_copies:
        cp.wait()

    def issue_in(w, slot):
        return [
            pltpu.async_copy(wte_hbm.at[idx_v.at[w]], tok_v.at[slot],
                             sem_in[slot]),
            pltpu.async_copy(wpe_hbm.at[pl.ds(p0 + w * WIN, WIN)],
                             pos_v.at[slot], sem_in[slot]),
        ]

    def issue_out(w, slot):
        return [
            pltpu.async_copy(tok_v.at[slot, pl.ds(b * WIN, WIN)],
                             out_hbm.at[b, pl.ds(p0 + w * WIN, WIN), :],
                             sem_out[slot])
            for b in range(BATCH)
        ]

    out_copies = [None] * DEPTH
    in_flight = {0: issue_in(0, 0)}
    for w in range(NWIN):
        slot = w % DEPTH
        for cp in in_flight.pop(w):
            cp.wait()

        def row_add(r):
            for k in range(EMB_VECS):
                sl = pl.ds(k * LANES, LANES)
                pv = pos_v[slot, r, sl]
                for b in range(BATCH):
                    plsc.addupdate(tok_v.at[slot, b * WIN + r, sl], pv)

        pl.loop(0, WIN // 2, unroll=1)(row_add)
        if w + 1 < NWIN:
            nslot = (w + 1) % DEPTH
            if out_copies[nslot] is not None:
                for cp in out_copies[nslot]:
                    cp.wait()
            in_flight[w + 1] = issue_in(w + 1, nslot)
        pl.loop(WIN // 2, WIN, unroll=1)(row_add)
        out_copies[slot] = issue_out(w, slot)
    for slot in range(DEPTH):
        if out_copies[slot] is not None:
            for cp in out_copies[slot]:
                cp.wait()


def kernel(input_ids, wte, wpe):
    ids = input_ids.astype(jnp.int32)
    return _embed_add(ids, wte, wpe)


# speedup vs baseline: 1.1821x; 1.1821x over previous
import functools

import jax
import jax.numpy as jnp
from jax import lax
from jax.experimental import pallas as pl
from jax.experimental.pallas import tpu as pltpu
from jax.experimental.pallas import tpu_sc as plsc

EMBED = 768
SEQ = 2048
BATCH = 4
NW = 32
POSW = SEQ // NW
WIN = 16
NWIN = POSW // WIN
LANES = 16
EMB_VECS = EMBED // LANES

_mesh = plsc.VectorSubcoreMesh(core_axis_name="c", subcore_axis_name="s")


@functools.partial(
    pl.kernel,
    out_type=jax.ShapeDtypeStruct((BATCH, SEQ, EMBED), jnp.float32),
    mesh=_mesh,
    scratch_types=[
        pltpu.VMEM((NWIN, BATCH * WIN), jnp.int32),
        pltpu.VMEM((2, BATCH * WIN, EMBED), jnp.float32),
        pltpu.VMEM((2, WIN, EMBED), jnp.float32),
        pltpu.SemaphoreType.DMA,
        (pltpu.SemaphoreType.DMA,) * 2,
        (pltpu.SemaphoreType.DMA,) * 2,
    ],
)
def _embed_add(ids_hbm, wte_hbm, wpe_hbm, out_hbm,
               idx_v, tok_v, pos_v, sem_idx, sem_in, sem_out):
    wid = lax.axis_index("s") * 2 + lax.axis_index("c")
    p0 = wid * POSW

    idx_copies = [
        pltpu.async_copy(ids_hbm.at[b, pl.ds(p0 + w * WIN, WIN)],
                         idx_v.at[w, pl.ds(b * WIN, WIN)], sem_idx)
        for w in range(NWIN) for b in range(BATCH)
    ]
    for cp in idx_copies:
        cp.wait()

    def in_descs(w, slot, make):
        mk = pltpu.make_async_copy if make else pltpu.async_copy
        return [
            mk(wte_hbm.at[idx_v.at[w]], tok_v.at[slot], sem_in[slot]),
            mk(wpe_hbm.at[pl.ds(p0 + w * WIN, WIN)], pos_v.at[slot],
               sem_in[slot]),
        ]

    def out_descs(w, slot, make):
        mk = pltpu.make_async_copy if make else pltpu.async_copy
        return [
            mk(tok_v.at[slot, pl.ds(b * WIN, WIN)],
               out_hbm.at[b, pl.ds(p0 + w * WIN, WIN), :], sem_out[slot])
            for b in range(BATCH)
        ]

    def wait_all(descs):
        for cp in descs:
            cp.wait()

    in_descs(0, 0, make=False)

    def pair(w0):
        w1 = w0 + 1
        wait_all(in_descs(w0, 0, make=True))

        @pl.when(w0 >= 1)
        def _drain_prev_b():
            wait_all(out_descs(w0 - 1, 1, make=True))

        in_descs(w1, 1, make=False)

        def row_add_a(r):
            for k in range(EMB_VECS):
                sl = pl.ds(k * LANES, LANES)
                pv = pos_v[0, r, sl]
                for b in range(BATCH):
                    plsc.addupdate(tok_v.at[0, b * WIN + r, sl], pv)

        pl.loop(0, WIN, unroll=1)(row_add_a)
        out_descs(w0, 0, make=False)

        wait_all(in_descs(w1, 1, make=True))

        @pl.when(w1 + 1 < NWIN)
        def _prefetch_next_a():
            wait_all(out_descs(w0, 0, make=True))
            in_descs(w1 + 1, 0, make=False)

        def row_add_b(r):
            for k in range(EMB_VECS):
                sl = pl.ds(k * LANES, LANES)
                pv = pos_v[1, r, sl]
                for b in range(BATCH):
                    plsc.addupdate(tok_v.at[1, b * WIN + r, sl], pv)

        pl.loop(0, WIN, unroll=1)(row_add_b)
        out_descs(w1, 1, make=False)

    pl.loop(0, NWIN, step=2, unroll=1)(pair)
    wait_all(out_descs(NWIN - 2, 0, make=True))
    wait_all(out_descs(NWIN - 1, 1, make=True))


def kernel(input_ids, wte, wpe):
    ids = input_ids.astype(jnp.int32)
    return _embed_add(ids, wte, wpe)


# speedup vs baseline: 1.1925x; 1.0088x over previous
import functools

import jax
import jax.numpy as jnp
from jax import lax
from jax.experimental import pallas as pl
from jax.experimental.pallas import tpu as pltpu
from jax.experimental.pallas import tpu_sc as plsc

EMBED = 768
SEQ = 2048
BATCH = 4
NW = 32
POSW = SEQ // NW
WIN = 8
NWIN = POSW // WIN
LANES = 16
EMB_VECS = EMBED // LANES

_mesh = plsc.VectorSubcoreMesh(core_axis_name="c", subcore_axis_name="s")


@functools.partial(
    pl.kernel,
    out_type=jax.ShapeDtypeStruct((BATCH, SEQ, EMBED), jnp.float32),
    mesh=_mesh,
    scratch_types=[
        pltpu.VMEM((NWIN, BATCH * WIN), jnp.int32),
        pltpu.VMEM((2, BATCH * WIN, EMBED), jnp.float32),
        pltpu.VMEM((2, BATCH * WIN, EMBED), jnp.float32),
        pltpu.VMEM((2, WIN, EMBED), jnp.float32),
        pltpu.SemaphoreType.DMA,
        (pltpu.SemaphoreType.DMA,) * 2,
        (pltpu.SemaphoreType.DMA,) * 2,
    ],
)
def _embed_add(ids_hbm, wte_hbm, wpe_hbm, out_hbm,
               idx_v, tok_v, stg_v, pos_v, sem_idx, sem_in, sem_out):
    wid = lax.axis_index("s") * 2 + lax.axis_index("c")
    p0 = wid * POSW

    idx_copies = [
        pltpu.async_copy(ids_hbm.at[b, pl.ds(p0 + w * WIN, WIN)],
                         idx_v.at[w, pl.ds(b * WIN, WIN)], sem_idx)
        for w in range(NWIN) for b in range(BATCH)
    ]
    for cp in idx_copies:
        cp.wait()

    def in_descs(w, slot, make):
        mk = pltpu.make_async_copy if make else pltpu.async_copy
        return [
            mk(wte_hbm.at[idx_v.at[w]], tok_v.at[slot], sem_in[slot]),
            mk(wpe_hbm.at[pl.ds(p0 + w * WIN, WIN)], pos_v.at[slot],
               sem_in[slot]),
        ]

    def out_descs(w, slot, make):
        mk = pltpu.make_async_copy if make else pltpu.async_copy
        return [
            mk(stg_v.at[slot, pl.ds(b * WIN, WIN)],
               out_hbm.at[b, pl.ds(p0 + w * WIN, WIN), :], sem_out[slot])
            for b in range(BATCH)
        ]

    def wait_all(descs):
        for cp in descs:
            cp.wait()

    in_descs(0, 0, make=False)

    def add_loop(slot):
        def row_add(r):
            for k in range(EMB_VECS):
                sl = pl.ds(k * LANES, LANES)
                pv = pos_v[slot, r, sl]
                for b in range(BATCH):
                    stg_v[slot, b * WIN + r, sl] = (
                        tok_v[slot, b * WIN + r, sl] + pv)

        pl.loop(0, WIN, unroll=1)(row_add)

    def step(w, slot, nslot):
        wait_all(in_descs(w, slot, make=True))

        @pl.when(w + 1 < NWIN)
        def _prefetch():
            in_descs(w + 1, nslot, make=False)

        @pl.when(w >= 2)
        def _drain():
            wait_all(out_descs(w - 2, slot, make=True))

        add_loop(slot)
        out_descs(w, slot, make=False)

    def pair(w0):
        step(w0, 0, 1)
        step(w0 + 1, 1, 0)

    pl.loop(0, NWIN, step=2, unroll=1)(pair)
    wait_all(out_descs(NWIN - 2, 0, make=True))
    wait_all(out_descs(NWIN - 1, 1, make=True))


def kernel(input_ids, wte, wpe):
    ids = input_ids.astype(jnp.int32)
    return _embed_add(ids, wte, wpe)


# speedup vs baseline: 1.2068x; 1.0120x over previous
import functools

import jax
import jax.numpy as jnp
from jax import lax
from jax.experimental import pallas as pl
from jax.experimental.pallas import tpu as pltpu
from jax.experimental.pallas import tpu_sc as plsc

EMBED = 768
SEQ = 2048
BATCH = 4
NW = 32
POSW = SEQ // NW
WIN = 8
NWIN = POSW // WIN
LANES = 16
EMB_VECS = EMBED // LANES

_mesh = plsc.VectorSubcoreMesh(core_axis_name="c", subcore_axis_name="s")


@functools.partial(
    pl.kernel,
    out_type=jax.ShapeDtypeStruct((BATCH, SEQ, EMBED), jnp.float32),
    mesh=_mesh,
    scratch_types=[
        pltpu.VMEM((NWIN, BATCH * WIN), jnp.int32),
        pltpu.VMEM((2, BATCH * WIN, EMBED), jnp.float32),
        pltpu.VMEM((2, BATCH * WIN, EMBED), jnp.float32),
        pltpu.VMEM((2, WIN, EMBED), jnp.float32),
        (pltpu.SemaphoreType.DMA,) * 2,
        (pltpu.SemaphoreType.DMA,) * 2,
        (pltpu.SemaphoreType.DMA,) * 2,
    ],
)
def _embed_add(ids_hbm, wte_hbm, wpe_hbm, out_hbm,
               idx_v, tok_v, stg_v, pos_v, sem_idx, sem_in, sem_out):
    wid = lax.axis_index("s") * 2 + lax.axis_index("c")
    p0 = wid * POSW

    idx_copies = [
        pltpu.async_copy(ids_hbm.at[b, pl.ds(p0 + w * WIN, WIN)],
                         idx_v.at[w, pl.ds(b * WIN, WIN)],
                         sem_idx[min(w, 1)])
        for w in range(NWIN) for b in range(BATCH)
    ]

    def in_descs(w, slot, make):
        mk = pltpu.make_async_copy if make else pltpu.async_copy
        return [
            mk(wte_hbm.at[idx_v.at[w]], tok_v.at[slot], sem_in[slot]),
            mk(wpe_hbm.at[pl.ds(p0 + w * WIN, WIN)], pos_v.at[slot],
               sem_in[slot]),
        ]

    def out_descs(w, slot, make):
        mk = pltpu.make_async_copy if make else pltpu.async_copy
        return [
            mk(stg_v.at[slot, pl.ds(b * WIN, WIN)],
               out_hbm.at[b, pl.ds(p0 + w * WIN, WIN), :], sem_out[slot])
            for b in range(BATCH)
        ]

    def wait_all(descs):
        for cp in descs:
            cp.wait()

    for cp in idx_copies[:BATCH]:
        cp.wait()
    in_descs(0, 0, make=False)
    for cp in idx_copies[BATCH:]:
        cp.wait()
    in_descs(1, 1, make=False)

    def add_loop(slot):
        def row_add(r):
            for k in range(EMB_VECS):
                sl = pl.ds(k * LANES, LANES)
                pv = pos_v[slot, r, sl]
                for b in range(BATCH):
                    stg_v[slot, b * WIN + r, sl] = (
                        tok_v[slot, b * WIN + r, sl] + pv)

        pl.loop(0, WIN, unroll=1)(row_add)

    def step(w, slot):
        wait_all(in_descs(w, slot, make=True))

        @pl.when(w >= 2)
        def _drain():
            wait_all(out_descs(w - 2, slot, make=True))

        add_loop(slot)

        @pl.when(w + 2 < NWIN)
        def _prefetch():
            in_descs(w + 2, slot, make=False)

        out_descs(w, slot, make=False)

    def pair(w0):
        step(w0, 0)
        step(w0 + 1, 1)

    pl.loop(0, NWIN, step=2, unroll=1)(pair)
    wait_all(out_descs(NWIN - 2, 0, make=True))
    wait_all(out_descs(NWIN - 1, 1, make=True))


def kernel(input_ids, wte, wpe):
    ids = input_ids.astype(jnp.int32)
    return _embed_add(ids, wte, wpe)


# speedup vs baseline: 1.2304x; 1.0196x over previous
import functools

import jax
import jax.numpy as jnp
from jax import lax
from jax.experimental import pallas as pl
from jax.experimental.pallas import tpu as pltpu
from jax.experimental.pallas import tpu_sc as plsc

EMBED = 768
SEQ = 2048
BATCH = 4
NW = 32
POSW = SEQ // NW
WIN = 8
NWIN = POSW // WIN
LANES = 16
EMB_VECS = EMBED // LANES

_mesh = plsc.VectorSubcoreMesh(core_axis_name="c", subcore_axis_name="s")


@functools.partial(
    pl.kernel,
    out_type=jax.ShapeDtypeStruct((BATCH, SEQ, EMBED), jnp.float32),
    mesh=_mesh,
    scratch_types=[
        pltpu.VMEM((NWIN, BATCH * WIN), jnp.int32),
        pltpu.VMEM((2, BATCH * WIN, EMBED), jnp.float32),
        pltpu.VMEM((2, BATCH * WIN, EMBED), jnp.float32),
        pltpu.VMEM((2, WIN, EMBED), jnp.float32),
        (pltpu.SemaphoreType.DMA,) * 2,
        (pltpu.SemaphoreType.DMA,) * 2,
        (pltpu.SemaphoreType.DMA,) * 2,
    ],
)
def _embed_add(ids_hbm, wte_hbm, wpe_hbm, out_hbm,
               idx_v, tok_v, stg_v, pos_v, sem_idx, sem_in, sem_out):
    wid = lax.axis_index("s") * 2 + lax.axis_index("c")
    p0 = wid * POSW

    idx_copies = [
        pltpu.async_copy(ids_hbm.at[b, pl.ds(p0 + w * WIN, WIN)],
                         idx_v.at[w, pl.ds(b * WIN, WIN)],
                         sem_idx[min(w, 1)])
        for w in range(NWIN) for b in range(BATCH)
    ]

    def in_descs(w, slot, make):
        mk = pltpu.make_async_copy if make else pltpu.async_copy
        return [
            mk(wte_hbm.at[idx_v.at[w]], tok_v.at[slot], sem_in[slot]),
            mk(wpe_hbm.at[pl.ds(p0 + w * WIN, WIN)], pos_v.at[slot],
               sem_in[slot]),
        ]

    def out_descs(w, slot, make):
        mk = pltpu.make_async_copy if make else pltpu.async_copy
        return [
            mk(stg_v.at[slot, pl.ds(b * WIN, WIN)],
               out_hbm.at[b, pl.ds(p0 + w * WIN, WIN), :], sem_out[slot])
            for b in range(BATCH)
        ]

    def wait_all(descs):
        for cp in descs:
            cp.wait()

    for cp in idx_copies[:BATCH]:
        cp.wait()
    in_descs(0, 0, make=False)
    for cp in idx_copies[BATCH:]:
        cp.wait()
    in_descs(1, 1, make=False)

    def add_loop(slot):
        def row_add(r):
            for k in range(EMB_VECS):
                sl = pl.ds(k * LANES, LANES)
                pv = pos_v[slot, r, sl]
                for b in range(BATCH):
                    stg_v[slot, b * WIN + r, sl] = (
                        tok_v[slot, b * WIN + r, sl] + pv)

        pl.loop(0, WIN, unroll=2)(row_add)

    def step(w, slot):
        wait_all(in_descs(w, slot, make=True))

        @pl.when(w >= 2)
        def _drain():
            wait_all(out_descs(w - 2, slot, make=True))

        add_loop(slot)

        @pl.when(w + 2 < NWIN)
        def _prefetch():
            in_descs(w + 2, slot, make=False)

        out_descs(w, slot, make=False)

    def pair(w0):
        step(w0, 0)
        step(w0 + 1, 1)

    pl.loop(0, NWIN, step=2, unroll=1)(pair)
    wait_all(out_descs(NWIN - 2, 0, make=True))
    wait_all(out_descs(NWIN - 1, 1, make=True))


def kernel(input_ids, wte, wpe):
    ids = input_ids.astype(jnp.int32)
    return _embed_add(ids, wte, wpe)
